# Initial kernel scaffold; baseline (speedup 1.0000x reference)
import jax
import jax.numpy as jnp
from jax.experimental import pallas as pl


def kernel(x, edge_index, W1, b1, ln_g, ln_b, Wc1, bc1, bn_g, bn_b, Wc2, bc2):
    raise NotImplementedError("write your pallas kernel here")


# speedup vs baseline: 20.1768x; 20.1768x over previous
import jax
import jax.numpy as jnp
from jax import lax
from jax.experimental import pallas as pl
from jax.experimental.pallas import tpu as pltpu
from jax.experimental.pallas import tpu_sc as plsc

_NC = 2
_NS = 16
_NW = _NC * _NS

_CH = 80
_RB = 2000


def _prolog_body(x_ref, w1t_ref, b1_ref, lng_ref, lnb_ref,
                 inp0_ref, g_ref, invr_ref):
    h = jnp.dot(x_ref[...], w1t_ref[...],
                preferred_element_type=jnp.float32) + b1_ref[...]
    s = jnp.maximum(h, 0.0)
    m = jnp.mean(s, axis=1, keepdims=True)
    v = jnp.mean((s - m) ** 2, axis=1, keepdims=True)
    inp0 = (s - m) / jnp.sqrt(v + 1e-5) * lng_ref[...] + lnb_ref[...]
    r = jnp.sqrt(jnp.sum(inp0 * inp0, axis=1, keepdims=True))
    invr = 1.0 / (r + 1e-4)
    inp0_ref[...] = inp0
    g_ref[...] = inp0 * invr
    invr_ref[...] = invr


def _prolog(x, w1t, b1, lng, lnb):
    n, d = x.shape
    grid = (n // _RB,)
    row_spec = pl.BlockSpec((_RB, d), lambda i: (i, 0))
    full_spec = pl.BlockSpec((d, d), lambda i: (0, 0))
    vec_spec = pl.BlockSpec((1, d), lambda i: (0, 0))
    return pl.pallas_call(
        _prolog_body,
        grid=grid,
        in_specs=[row_spec, full_spec, vec_spec, vec_spec, vec_spec],
        out_specs=[row_spec, row_spec, pl.BlockSpec((_RB, 1), lambda i: (i, 0))],
        out_shape=[
            jax.ShapeDtypeStruct((n, d), jnp.float32),
            jax.ShapeDtypeStruct((n, d), jnp.float32),
            jax.ShapeDtypeStruct((n, 1), jnp.float32),
        ],
    )(x, w1t, b1, lng, lnb)


def _scale_body(d0_ref, d1_ref, g_ref, dinv_ref, gs_ref):
    deg = d0_ref[:, 0:1] + d1_ref[:, 0:1]
    dinv = jnp.where(deg > 0, 1.0 / jnp.sqrt(deg), 0.0)
    dinv_ref[...] = dinv
    gs_ref[...] = g_ref[...] * dinv


def _scale(d0, d1, g):
    n, d = g.shape
    grid = (n // _RB,)
    return pl.pallas_call(
        _scale_body,
        grid=grid,
        in_specs=[pl.BlockSpec((_RB, 8), lambda i: (i, 0)),
                  pl.BlockSpec((_RB, 8), lambda i: (i, 0)),
                  pl.BlockSpec((_RB, d), lambda i: (i, 0))],
        out_specs=[pl.BlockSpec((_RB, 1), lambda i: (i, 0)),
                   pl.BlockSpec((_RB, d), lambda i: (i, 0))],
        out_shape=[jax.ShapeDtypeStruct((n, 1), jnp.float32),
                   jax.ShapeDtypeStruct((n, d), jnp.float32)],
    )(d0, d1, g)


def _layer1_body(g_ref, a0_ref, a1_ref, inp0_ref, dinv_ref, invr_ref,
                 gs2_ref, p2_ref, t_ref):
    g = g_ref[...]
    agg = a0_ref[...] + a1_ref[...]
    inp0 = inp0_ref[...]
    dinv = dinv_ref[...]
    invr = invr_ref[...]
    prod = g * agg
    full = jnp.sum(prod, axis=1, keepdims=True)
    li = lax.broadcasted_iota(jnp.int32, prod.shape, 1)
    t = full - jnp.sum(jnp.where(li < 2, prod, 0.0), axis=1, keepdims=True)
    ang = dinv * full
    c = jnp.cos(ang)
    s = jnp.sin(ang)
    p0 = jnp.sum(jnp.where(li == 0, inp0, 0.0), axis=1, keepdims=True)
    p1 = jnp.sum(jnp.where(li == 1, inp0, 0.0), axis=1, keepdims=True)
    q0 = p0 * c - p1 * s
    q1 = p0 * s + p1 * c
    sc = invr * dinv
    li8 = lax.broadcasted_iota(jnp.int32, (gs2_ref.shape[0], 8), 1)
    gs2_ref[...] = jnp.where(li8 == 0, q0 * sc,
                             jnp.where(li8 == 1, q1 * sc, 0.0))
    li2 = lax.broadcasted_iota(jnp.int32, (p2_ref.shape[0], 2), 1)
    p2_ref[...] = jnp.where(li2 == 0, q0, q1)
    t_ref[...] = t


def _layer1(g, a0, a1, inp0, dinv, invr):
    n, d = g.shape
    grid = (n // _RB,)
    wide = pl.BlockSpec((_RB, d), lambda i: (i, 0))
    one = pl.BlockSpec((_RB, 1), lambda i: (i, 0))
    return pl.pallas_call(
        _layer1_body,
        grid=grid,
        in_specs=[wide, wide, wide, wide, one, one],
        out_specs=[pl.BlockSpec((_RB, 8), lambda i: (i, 0)),
                   pl.BlockSpec((_RB, 2), lambda i: (i, 0)),
                   one],
        out_shape=[jax.ShapeDtypeStruct((n, 8), jnp.float32),
                   jax.ShapeDtypeStruct((n, 2), jnp.float32),
                   jax.ShapeDtypeStruct((n, 1), jnp.float32)],
    )(g, a0, a1, inp0, dinv, invr)


def _final_body(ap0_ref, ap1_ref, p2_ref, t_ref, dinv_ref, invr_ref, inp0_ref,
                wc1t_ref, bc1_ref, bng_ref, bnb_ref, wc2t_ref, bc2_ref,
                out_ref):
    aggp = ap0_ref[...] + ap1_ref[...]
    p2 = p2_ref[...]
    dinv = dinv_ref[...]
    invr = invr_ref[...]
    nb = p2.shape[0]
    li2 = lax.broadcasted_iota(jnp.int32, (nb, 2), 1)
    p20 = jnp.sum(jnp.where(li2 == 0, p2, 0.0), axis=1, keepdims=True)
    p21 = jnp.sum(jnp.where(li2 == 1, p2, 0.0), axis=1, keepdims=True)
    li8 = lax.broadcasted_iota(jnp.int32, (nb, 8), 1)
    ap0 = jnp.sum(jnp.where(li8 == 0, aggp, 0.0), axis=1, keepdims=True)
    ap1 = jnp.sum(jnp.where(li8 == 1, aggp, 0.0), axis=1, keepdims=True)
    pdot = invr * (p20 * ap0 + p21 * ap1)
    ang = dinv * (t_ref[...] + pdot)
    c = jnp.cos(ang)
    s = jnp.sin(ang)
    r0 = p20 * c - p21 * s
    r1 = p20 * s + p21 * c
    inp0 = inp0_ref[...]
    li = lax.broadcasted_iota(jnp.int32, inp0.shape, 1)
    inp3 = jnp.where(li == 0, r0, jnp.where(li == 1, r1, inp0))
    h = jnp.dot(inp3, wc1t_ref[...], preferred_element_type=jnp.float32)
    h = jnp.maximum(h + bc1_ref[...], 0.0)
    h = h * (bng_ref[...] * (1.0 / jnp.sqrt(1.0 + 1e-5))) + bnb_ref[...]
    o = jnp.dot(h, wc2t_ref[...], preferred_element_type=jnp.float32)
    o = o + bc2_ref[...]
    mx = jnp.max(o, axis=1, keepdims=True)
    z = o - mx
    lse = jnp.log(jnp.sum(jnp.exp(z), axis=1, keepdims=True))
    out_ref[...] = z - lse


def _final(ap0, ap1, p2, t, dinv, invr, inp0, wc1t, bc1, bng, bnb, wc2t, bc2):
    n, d = inp0.shape
    dout = wc2t.shape[1]
    grid = (n // _RB,)
    one = pl.BlockSpec((_RB, 1), lambda i: (i, 0))
    eight = pl.BlockSpec((_RB, 8), lambda i: (i, 0))
    wide = pl.BlockSpec((_RB, d), lambda i: (i, 0))
    vec = pl.BlockSpec((1, d), lambda i: (0, 0))
    return pl.pallas_call(
        _final_body,
        grid=grid,
        in_specs=[eight, eight, pl.BlockSpec((_RB, 2), lambda i: (i, 0)),
                  one, one, one, wide,
                  pl.BlockSpec((d, d), lambda i: (0, 0)), vec, vec, vec,
                  pl.BlockSpec((d, dout), lambda i: (0, 0)),
                  pl.BlockSpec((1, dout), lambda i: (0, 0))],
        out_specs=[pl.BlockSpec((_RB, dout), lambda i: (i, 0))],
        out_shape=[jax.ShapeDtypeStruct((n, dout), jnp.float32)],
    )(ap0, ap1, p2, t, dinv, invr, inp0, wc1t, bc1, bng, bnb, wc2t, bc2)[0]


def _sc_mesh():
    return plsc.VectorSubcoreMesh(core_axis_name="c", subcore_axis_name="s")


def _sc_deg(row3, ones_tile, zero8):
    n = zero8.shape[0]
    nch = row3.shape[1]
    stripe = n // _NS

    def body(row_hbm, ones_hbm, zero_hbm, out_hbm, idx_v, ones_v, acc_sh):
        cid = lax.axis_index("c")
        sid = lax.axis_index("s")
        wid = sid * _NC + cid
        pltpu.sync_copy(row_hbm.at[wid], idx_v)
        pltpu.sync_copy(ones_hbm, ones_v)
        pltpu.sync_copy(zero_hbm.at[pl.ds(sid * stripe, stripe)],
                        acc_sh.at[pl.ds(sid * stripe, stripe)])
        plsc.subcore_barrier()

        def step(j, carry):
            pltpu.sync_copy(ones_v, acc_sh.at[idx_v.at[j]], add=True)
            return carry

        lax.fori_loop(0, nch, step, 0)
        plsc.subcore_barrier()
        pltpu.sync_copy(acc_sh.at[pl.ds(sid * stripe, stripe)],
                        out_hbm.at[cid, pl.ds(sid * stripe, stripe)])

    return pl.kernel(
        body,
        out_type=jax.ShapeDtypeStruct((_NC, n, 8), jnp.float32),
        mesh=_sc_mesh(),
        scratch_types=[
            pltpu.VMEM((nch, _CH), jnp.int32),
            pltpu.VMEM((_CH, 8), jnp.float32),
            pltpu.VMEM_SHARED((n, 8), jnp.float32),
        ],
    )(row3, ones_tile, zero8)


def _sc_spmm(row3, col3, table, zeros, stage_table=False):
    w = table.shape[1]
    npad = zeros.shape[0]
    nch = row3.shape[1]
    stripe = npad // _NS

    def body(row_hbm, col_hbm, tab_hbm, zero_hbm, out_hbm,
             ridx_v, cidx_v, rows_v, acc_sh, tab_sh, sem):
        cid = lax.axis_index("c")
        sid = lax.axis_index("s")
        wid = sid * _NC + cid
        pltpu.sync_copy(row_hbm.at[wid], ridx_v)
        pltpu.sync_copy(col_hbm.at[wid], cidx_v)
        pltpu.sync_copy(zero_hbm.at[pl.ds(sid * stripe, stripe)],
                        acc_sh.at[pl.ds(sid * stripe, stripe)])
        if stage_table:
            pltpu.sync_copy(tab_hbm.at[pl.ds(sid * stripe, stripe)],
                            tab_sh.at[pl.ds(sid * stripe, stripe)])
        plsc.subcore_barrier()

        src = tab_sh if stage_table else tab_hbm

        def step(j, carry):
            pltpu.async_copy(src.at[cidx_v.at[j]], rows_v, sem).wait()
            pltpu.sync_copy(rows_v, acc_sh.at[ridx_v.at[j]], add=True)
            return carry

        lax.fori_loop(0, nch, step, 0)
        plsc.subcore_barrier()
        pltpu.sync_copy(acc_sh.at[pl.ds(sid * stripe, stripe)],
                        out_hbm.at[cid, pl.ds(sid * stripe, stripe)])

    tab_sh_shape = (npad, w) if stage_table else (8, w)
    return pl.kernel(
        body,
        out_type=jax.ShapeDtypeStruct((_NC, npad, w), jnp.float32),
        mesh=_sc_mesh(),
        scratch_types=[
            pltpu.VMEM((nch, _CH), jnp.int32),
            pltpu.VMEM((nch, _CH), jnp.int32),
            pltpu.VMEM((_CH, w), jnp.float32),
            pltpu.VMEM_SHARED((npad, w), jnp.float32),
            pltpu.VMEM_SHARED(tab_sh_shape, jnp.float32),
            pltpu.SemaphoreType.DMA,
        ],
    )(row3, col3, table, zeros)


def kernel(x, edge_index, W1, b1, ln_g, ln_b, Wc1, bc1, bn_g, bn_b, Wc2, bc2):
    n, d = x.shape
    e = edge_index.shape[1]
    assert e % (_NW * _CH) == 0 and n % _RB == 0
    nch = e // (_NW * _CH)
    npad = ((n + 8 * _NS - 1) // (8 * _NS)) * (8 * _NS)

    row3 = edge_index[0].reshape(_NW, nch, _CH)
    col3 = edge_index[1].reshape(_NW, nch, _CH)

    ones_tile = jnp.ones((_CH, 8), jnp.float32)
    zero8 = jnp.zeros((npad, 8), jnp.float32)
    zero128 = jnp.zeros((npad, d), jnp.float32)

    inp0, g, invr = _prolog(x, W1.T, b1[None], ln_g[None], ln_b[None])
    deg_parts = _sc_deg(row3, ones_tile, zero8)
    dinv, gs = _scale(deg_parts[0, :n], deg_parts[1, :n], g)
    agg_parts = _sc_spmm(row3, col3, gs, zero128)
    gs2, p2, t = _layer1(g, agg_parts[0, :n], agg_parts[1, :n], inp0,
                         dinv, invr)
    gs2p = jnp.pad(gs2, ((0, npad - n), (0, 0)))
    aggp = _sc_spmm(row3, col3, gs2p, zero8, stage_table=True)
    return _final(aggp[0, :n], aggp[1, :n], p2, t, dinv, invr, inp0,
                  Wc1.T, bc1[None], bn_g[None], bn_b[None], Wc2.T, bc2[None])
